# baseline (speedup 1.0000x reference)
import functools

import jax
import jax.numpy as jnp
from jax import lax
from jax.experimental import pallas as pl
from jax.experimental.pallas import tpu as pltpu
from jax.experimental.pallas import tpu_sc as plsc

B, S, F = 1024, 512, 128
NW = 32
EPW = B // NW
NV = F // 16
NEG = -1e30
U = 8
CHUNKS = ((0, 176), (176, 176), (352, 160))

_mesh = plsc.VectorSubcoreMesh(
    core_axis_name="c", subcore_axis_name="s", num_cores=2, num_subcores=16
)


def _pure_block_loop(buf, lo, hi, acc):
    def body(bi, acc):
        r0 = bi * U
        for j in range(U):
            row = [buf[r0 + j, pl.ds(v * 16, 16)] for v in range(NV)]
            acc = tuple(jnp.maximum(acc[v], row[v]) for v in range(NV))
        return acc

    return lax.fori_loop(lo, hi, body, acc)


def _masked_block(buf, mb, c0, e1s, e2s, accs):
    inf = jnp.float32(jnp.inf)
    ninf = jnp.float32(-jnp.inf)
    a1, a2, a3 = accs
    r0 = mb * U
    for j in range(U):
        grow = c0 + r0 + j
        in1 = grow <= e1s
        in2 = (grow > e1s) & (grow <= e2s)
        in3 = grow > e2s
        c1 = jnp.broadcast_to(jnp.where(in1, inf, ninf), (16,))
        c2 = jnp.broadcast_to(jnp.where(in2, inf, ninf), (16,))
        c3 = jnp.broadcast_to(jnp.where(in3, inf, ninf), (16,))
        row = [buf[r0 + j, pl.ds(v * 16, 16)] for v in range(NV)]
        a1 = tuple(jnp.maximum(a1[v], jnp.minimum(row[v], c1)) for v in range(NV))
        a2 = tuple(jnp.maximum(a2[v], jnp.minimum(row[v], c2)) for v in range(NV))
        a3 = tuple(jnp.maximum(a3[v], jnp.minimum(row[v], c3)) for v in range(NV))
    return [a1, a2, a3]


@functools.partial(
    pl.kernel,
    out_type=jax.ShapeDtypeStruct((B, 3 * F), jnp.float32),
    mesh=_mesh,
    scratch_types=[
        pltpu.VMEM((3, 176, F), jnp.float32),
        pltpu.VMEM((EPW, 3 * F), jnp.float32),
        pltpu.VMEM((EPW, 16), jnp.int32),
        pltpu.SemaphoreType.DMA,
        pltpu.SemaphoreType.DMA,
        pltpu.SemaphoreType.DMA,
    ],
)
def _sc_piecewise_max(conv_hbm, ee_hbm, out_hbm, buf, out_v, e_v, sem0, sem1, sem2):
    wid = lax.axis_index("c") * 16 + lax.axis_index("s")
    base = wid * EPW

    pltpu.sync_copy(ee_hbm.at[pl.ds(base, EPW)], e_v)

    sems = (sem0, sem1, sem2)

    def dma(ex, q):
        c0, rows = CHUNKS[q]
        return pltpu.make_async_copy(
            conv_hbm.at[base + ex, pl.ds(c0, rows)],
            buf.at[q, pl.ds(0, rows)],
            sems[q],
        )

    dma(0, 0).start()
    dma(0, 1).start()

    def ex_body(i, carry):
        evec = e_v[i]
        e1s = evec[0]
        e2s = evec[1]
        neg = jnp.full((16,), NEG, jnp.float32)
        accs = [tuple(neg for _ in range(NV)) for _ in range(3)]
        for q in range(3):
            c0, rows = CHUNKS[q]
            nb = rows // U
            dma(i, q).wait()
            if q == 0:
                dma(i, 2).start()
            else:

                @pl.when(i + 1 < EPW)
                def _():
                    dma(i + 1, q - 1).start()

            cbuf = buf.at[q]
            a = jnp.clip(e1s + 1 - c0, 0, rows)
            b = jnp.clip(e2s + 1 - c0, 0, rows)
            ba = a // U
            bb = b // U
            accs[0] = _pure_block_loop(cbuf, 0, ba, accs[0])
            accs[1] = _pure_block_loop(cbuf, ba + 1, bb, accs[1])
            accs[2] = _pure_block_loop(cbuf, bb + 1, nb, accs[2])
            accs = _masked_block(cbuf, jnp.minimum(ba, nb - 1), c0, e1s, e2s, accs)
            accs = _masked_block(cbuf, jnp.minimum(bb, nb - 1), c0, e1s, e2s, accs)

        for p in range(3):
            for v in range(NV):
                out_v[i, pl.ds(p * F + v * 16, 16)] = accs[p][v]
        return carry

    lax.fori_loop(0, EPW, ex_body, 0)
    pltpu.sync_copy(out_v, out_hbm.at[pl.ds(base, EPW)])


def kernel(conv_output, e1, e2):
    ee = jnp.concatenate(
        [e1.astype(jnp.int32), e2.astype(jnp.int32)], axis=1
    )
    ee = jnp.pad(ee, ((0, 0), (0, 14)))
    return _sc_piecewise_max(conv_output, ee)


# speedup vs baseline: 1.9012x; 1.9012x over previous
import functools

import jax
import jax.numpy as jnp
from jax import lax
from jax.experimental import pallas as pl
from jax.experimental.pallas import tpu as pltpu
from jax.experimental.pallas import tpu_sc as plsc

B, S, F = 1024, 512, 128
NW = 32
EPW = B // NW
NV = F // 16
NEG = -1e30
CHUNKS = ((0, 176), (176, 176), (352, 160))

_mesh = plsc.VectorSubcoreMesh(
    core_axis_name="c", subcore_axis_name="s", num_cores=2, num_subcores=16
)


def _row_loop(buf, lo, hi, acc):
    def body(r, acc):
        return tuple(
            jnp.maximum(acc[v], buf[r, pl.ds(v * 16, 16)]) for v in range(NV)
        )

    return lax.fori_loop(lo, hi, body, acc)


@functools.partial(
    pl.kernel,
    out_type=jax.ShapeDtypeStruct((B, 3 * F), jnp.float32),
    mesh=_mesh,
    scratch_types=[
        pltpu.VMEM((3, 176, F), jnp.float32),
        pltpu.VMEM((EPW, 3 * F), jnp.float32),
        pltpu.VMEM((EPW, 16), jnp.int32),
        pltpu.SemaphoreType.DMA,
        pltpu.SemaphoreType.DMA,
        pltpu.SemaphoreType.DMA,
    ],
)
def _sc_piecewise_max(conv_hbm, ee_hbm, out_hbm, buf, out_v, e_v, sem0, sem1, sem2):
    wid = lax.axis_index("c") * 16 + lax.axis_index("s")
    base = wid * EPW

    pltpu.sync_copy(ee_hbm.at[pl.ds(base, EPW)], e_v)

    sems = (sem0, sem1, sem2)

    def dma(ex, q):
        c0, rows = CHUNKS[q]
        return pltpu.make_async_copy(
            conv_hbm.at[base + ex, pl.ds(c0, rows)],
            buf.at[q, pl.ds(0, rows)],
            sems[q],
        )

    dma(0, 0).start()
    dma(0, 1).start()

    def ex_body(i, carry):
        evec = e_v[i]
        e1s = evec[0]
        e2s = evec[1]
        neg = jnp.full((16,), NEG, jnp.float32)
        accs = [tuple(neg for _ in range(NV)) for _ in range(3)]
        for q in range(3):
            c0, rows = CHUNKS[q]
            dma(i, q).wait()
            if q == 0:
                dma(i, 2).start()
            else:

                @pl.when(i + 1 < EPW)
                def _():
                    dma(i + 1, q - 1).start()

            cbuf = buf.at[q]
            a = jnp.clip(e1s + 1 - c0, 0, rows)
            b = jnp.clip(e2s + 1 - c0, 0, rows)
            accs[0] = _row_loop(cbuf, 0, a, accs[0])
            accs[1] = _row_loop(cbuf, a, b, accs[1])
            accs[2] = _row_loop(cbuf, b, rows, accs[2])

        for p in range(3):
            for v in range(NV):
                out_v[i, pl.ds(p * F + v * 16, 16)] = accs[p][v]
        return carry

    lax.fori_loop(0, EPW, ex_body, 0)
    pltpu.sync_copy(out_v, out_hbm.at[pl.ds(base, EPW)])


def kernel(conv_output, e1, e2):
    ee = jnp.concatenate(
        [e1.astype(jnp.int32), e2.astype(jnp.int32)], axis=1
    )
    ee = jnp.pad(ee, ((0, 0), (0, 14)))
    return _sc_piecewise_max(conv_output, ee)
